# Initial kernel scaffold; baseline (speedup 1.0000x reference)
import jax
import jax.numpy as jnp
from jax.experimental import pallas as pl


def kernel(x, conv_w):
    raise NotImplementedError("write your pallas kernel here")


# speedup vs baseline: 1.0701x; 1.0701x over previous
import functools

import jax
import jax.numpy as jnp
from jax.experimental import pallas as pl
from jax.experimental.pallas import tpu as pltpu


def _eca_block_kernel(w_ref, x_ref, o_ref, *, k, c, inv_hw):
    xv = x_ref[...]
    s = jnp.sum(xv, axis=-1, keepdims=True, dtype=jnp.float32) * inv_hw
    b = s.shape[0]
    pad = k // 2
    conv = w_ref[pad] * s
    for j in range(k):
        d = j - pad
        if d == 0:
            continue
        if d < 0:
            z = jnp.zeros((b, -d, 1), jnp.float32)
            sh = jnp.concatenate([z, s[:, : c + d, :]], axis=1)
        else:
            z = jnp.zeros((b, d, 1), jnp.float32)
            sh = jnp.concatenate([s[:, d:, :], z], axis=1)
        conv = conv + w_ref[j] * sh
    attn = jax.nn.sigmoid(conv)
    o_ref[...] = (attn * xv).astype(o_ref.dtype)


def _pick_batch_block(n, bytes_per_image, budget):
    for b in (8, 4, 2, 1):
        if n % b == 0 and 4 * b * bytes_per_image <= budget:
            return b
    return 1


def kernel(x, conv_w):
    n, c, h, w = x.shape
    hw = h * w
    k = int(conv_w.shape[0])
    assert k % 2 == 1
    itemsize = jnp.dtype(x.dtype).itemsize
    inv_hw = 1.0 / float(hw)
    conv_w = conv_w.reshape(-1).astype(jnp.float32)
    x_flat = x.reshape(n, c, hw)

    budget = 52 * 1024 * 1024
    bpi = c * hw * itemsize
    b = _pick_batch_block(n, bpi, budget)

    out_flat = pl.pallas_call(
        functools.partial(_eca_block_kernel, k=k, c=c, inv_hw=inv_hw),
        out_shape=jax.ShapeDtypeStruct((n, c, hw), x.dtype),
        grid=(n // b,),
        in_specs=[
            pl.BlockSpec(memory_space=pltpu.MemorySpace.SMEM),
            pl.BlockSpec((b, c, hw), lambda i: (i, 0, 0)),
        ],
        out_specs=pl.BlockSpec((b, c, hw), lambda i: (i, 0, 0)),
        compiler_params=pltpu.CompilerParams(
            dimension_semantics=("parallel",),
            vmem_limit_bytes=60 * 1024 * 1024),
        cost_estimate=pl.CostEstimate(
            flops=2 * n * c * hw + 2 * n * c * k,
            transcendentals=n * c,
            bytes_accessed=2 * n * c * hw * itemsize),
    )(conv_w, x_flat)

    return out_flat.reshape(n, c, h, w)
